# Initial kernel scaffold; baseline (speedup 1.0000x reference)
import jax
import jax.numpy as jnp
from jax.experimental import pallas as pl


def kernel(x, edge_index, batch_idx, edge_attr, W_emb, b_emb, Wm0, bm0, Wu0, bu0, g0, be0, Wm1, bm1, Wu1, bu1, g1, be1, Wo1, bo1, Wo2, bo2):
    raise NotImplementedError("write your pallas kernel here")


# speedup vs baseline: 1.7517x; 1.7517x over previous
import functools

import jax
import jax.numpy as jnp
from jax import lax
from jax.experimental import pallas as pl
from jax.experimental.pallas import tpu as pltpu
from jax.experimental.pallas import tpu_sc as plsc

N = 10000
E = 160000
DF = 256
H = 512
DE = 16
OUT = 128
G = 64
EPS = 1e-5

F32 = jnp.float32

NB = 1000
NGRID = N // NB
EB = 1000
EGRID = E // EB

NC = 2
NS = 16
NW = NC * NS


def _embed_body(x_ref, w_ref, b_ref, o_ref):
    o_ref[...] = (
        jnp.dot(x_ref[...], w_ref[...], preferred_element_type=F32) + b_ref[...]
    )


def _embed(x, w, b):
    return pl.pallas_call(
        _embed_body,
        grid=(NGRID,),
        in_specs=[
            pl.BlockSpec((NB, DF), lambda i: (i, 0)),
            pl.BlockSpec((DF, H), lambda i: (0, 0)),
            pl.BlockSpec((1, H), lambda i: (0, 0)),
        ],
        out_specs=pl.BlockSpec((NB, H), lambda i: (i, 0)),
        out_shape=jax.ShapeDtypeStruct((N, H), F32),
    )(x, w, b.reshape(1, H))


def _proj3_body(h_ref, w_ref, a_ref, b_ref, u_ref):
    r = jnp.dot(h_ref[...], w_ref[...], preferred_element_type=F32)
    a_ref[...] = r[:, :H]
    b_ref[...] = r[:, H : 2 * H]
    u_ref[...] = r[:, 2 * H :]


def _proj3(h, wcat):
    return pl.pallas_call(
        _proj3_body,
        grid=(NGRID,),
        in_specs=[
            pl.BlockSpec((NB, H), lambda i: (i, 0)),
            pl.BlockSpec((H, 3 * H), lambda i: (0, 0)),
        ],
        out_specs=[
            pl.BlockSpec((NB, H), lambda i: (i, 0)),
            pl.BlockSpec((NB, H), lambda i: (i, 0)),
            pl.BlockSpec((NB, H), lambda i: (i, 0)),
        ],
        out_shape=[
            jax.ShapeDtypeStruct((N, H), F32),
            jax.ShapeDtypeStruct((N, H), F32),
            jax.ShapeDtypeStruct((N, H), F32),
        ],
    )(h, wcat)


def _msg_body(g1_ref, g2_ref, ea_ref, wb_ref, bm_ref, m0, m1, m2, m3):
    m = g1_ref[...] + g2_ref[...]
    m = m + jnp.dot(ea_ref[...], wb_ref[...], preferred_element_type=F32)
    m = jnp.maximum(m + bm_ref[...], 0.0)
    m0[...] = m[:, 0:128]
    m1[...] = m[:, 128:256]
    m2[...] = m[:, 256:384]
    m3[...] = m[:, 384:512]


def _msg(g1, g2, ea, wb, bm):
    strip = jax.ShapeDtypeStruct((E, 128), F32)
    return pl.pallas_call(
        _msg_body,
        grid=(EGRID,),
        in_specs=[
            pl.BlockSpec((EB, H), lambda i: (i, 0)),
            pl.BlockSpec((EB, H), lambda i: (i, 0)),
            pl.BlockSpec((EB, DE), lambda i: (i, 0)),
            pl.BlockSpec((DE, H), lambda i: (0, 0)),
            pl.BlockSpec((1, H), lambda i: (0, 0)),
        ],
        out_specs=[pl.BlockSpec((EB, 128), lambda i: (i, 0)) for _ in range(4)],
        out_shape=[strip, strip, strip, strip],
    )(g1, g2, ea, wb, bm.reshape(1, H))


def _update_body(u_ref, a0, a1, a2, a3, wub_ref, bu_ref, g_ref, be_ref, hp_ref, o_ref):
    aggr = jnp.concatenate([a0[...], a1[...], a2[...], a3[...]], axis=-1)
    t = (
        u_ref[...]
        + jnp.dot(aggr, wub_ref[...], preferred_element_type=F32)
        + bu_ref[...]
    )
    mu = jnp.mean(t, axis=-1, keepdims=True)
    var = jnp.mean((t - mu) * (t - mu), axis=-1, keepdims=True)
    o_ref[...] = (t - mu) * lax.rsqrt(var + EPS) * g_ref[...] + be_ref[...] + hp_ref[...]


def _update(u, aggr_strips, wub, bu, g, be, h_prev):
    a0, a1, a2, a3 = aggr_strips
    return pl.pallas_call(
        _update_body,
        grid=(NGRID,),
        in_specs=[
            pl.BlockSpec((NB, H), lambda i: (i, 0)),
            pl.BlockSpec((NB, 128), lambda i: (i, 0)),
            pl.BlockSpec((NB, 128), lambda i: (i, 0)),
            pl.BlockSpec((NB, 128), lambda i: (i, 0)),
            pl.BlockSpec((NB, 128), lambda i: (i, 0)),
            pl.BlockSpec((H, H), lambda i: (0, 0)),
            pl.BlockSpec((1, H), lambda i: (0, 0)),
            pl.BlockSpec((1, H), lambda i: (0, 0)),
            pl.BlockSpec((1, H), lambda i: (0, 0)),
            pl.BlockSpec((NB, H), lambda i: (i, 0)),
        ],
        out_specs=pl.BlockSpec((NB, H), lambda i: (i, 0)),
        out_shape=jax.ShapeDtypeStruct((N, H), F32),
    )(u, a0, a1, a2, a3, wub, bu.reshape(1, H), g.reshape(1, H), be.reshape(1, H), h_prev)


def _pool_body(h_ref, bidx_ref, o_ref):
    i = pl.program_id(0)
    b = bidx_ref[0, 0, :]
    oh = (lax.broadcasted_iota(jnp.int32, (G, NB), 0) == b[None, :]).astype(F32)
    part = jnp.dot(oh, h_ref[...], preferred_element_type=F32)

    @pl.when(i == 0)
    def _():
        o_ref[...] = part

    @pl.when(i > 0)
    def _():
        o_ref[...] += part


def _pool(h, batch_idx):
    bidx3 = batch_idx.reshape(NGRID, 1, NB)
    return pl.pallas_call(
        _pool_body,
        grid=(NGRID,),
        in_specs=[
            pl.BlockSpec((NB, H), lambda i: (i, 0)),
            pl.BlockSpec((1, 1, NB), lambda i: (i, 0, 0)),
        ],
        out_specs=pl.BlockSpec((G, H), lambda i: (0, 0)),
        out_shape=jax.ShapeDtypeStruct((G, H), F32),
    )(h, bidx3)


def _head_body(p_ref, w1_ref, b1_ref, w2_ref, b2_ref, o_ref):
    t = jnp.dot(p_ref[...], w1_ref[...], preferred_element_type=F32) + b1_ref[...]
    o_ref[...] = jnp.dot(t, w2_ref[...], preferred_element_type=F32) + b2_ref[...]


def _head(p, w1, b1, w2, b2):
    return pl.pallas_call(
        _head_body,
        in_specs=[
            pl.BlockSpec((G, H), lambda: (0, 0)),
            pl.BlockSpec((H, H), lambda: (0, 0)),
            pl.BlockSpec((1, H), lambda: (0, 0)),
            pl.BlockSpec((H, OUT), lambda: (0, 0)),
            pl.BlockSpec((1, OUT), lambda: (0, 0)),
        ],
        out_specs=pl.BlockSpec((G, OUT), lambda: (0, 0)),
        out_shape=jax.ShapeDtypeStruct((G, OUT), F32),
    )(p, w1, b1.reshape(1, H), w2, b2.reshape(1, OUT))


EPW = E // NW
GCH = 120
GNCH = 41
GTAIL = EPW - GNCH * GCH


@functools.lru_cache(maxsize=None)
def _sc_mesh():
    return plsc.VectorSubcoreMesh(core_axis_name="c", subcore_axis_name="s")


@functools.lru_cache(maxsize=None)
def _gather_ab_kernel():
    @functools.partial(
        pl.kernel,
        mesh=_sc_mesh(),
        out_type=(
            jax.ShapeDtypeStruct((E, H), F32),
            jax.ShapeDtypeStruct((E, H), F32),
        ),
        scratch_types=[
            pltpu.VMEM((GCH,), jnp.int32),
            pltpu.VMEM((GTAIL,), jnp.int32),
            pltpu.VMEM((GCH, H), F32),
        ],
    )
    def _gather_ab(a_hbm, b_hbm, src_hbm, dst_hbm, g1_hbm, g2_hbm, idx_v, idxt_v, rows_v):
        wid = lax.axis_index("s") * NC + lax.axis_index("c")
        base = wid * EPW

        def one_table(tab, idx_arr, out):
            @pl.loop(0, GNCH)
            def _(k):
                off = base + k * GCH
                pltpu.sync_copy(idx_arr.at[pl.ds(off, GCH)], idx_v)
                pltpu.sync_copy(tab.at[idx_v], rows_v)
                pltpu.sync_copy(rows_v, out.at[pl.ds(off, GCH)])

            off = base + GNCH * GCH
            pltpu.sync_copy(idx_arr.at[pl.ds(off, GTAIL)], idxt_v)
            pltpu.sync_copy(tab.at[idxt_v], rows_v.at[pl.ds(0, GTAIL)])
            pltpu.sync_copy(rows_v.at[pl.ds(0, GTAIL)], out.at[pl.ds(off, GTAIL)])

        one_table(a_hbm, src_hbm, g1_hbm)
        one_table(b_hbm, dst_hbm, g2_hbm)

    return _gather_ab


def _gather_ab(a, b, src, dst):
    return _gather_ab_kernel()(a, b, src, dst)


EPS_SC = E // NS
SCH = 80
SNCH = EPS_SC // SCH
ZROWS = 640


@functools.lru_cache(maxsize=None)
def _scatter_add_kernel():
    @functools.partial(
        pl.kernel,
        mesh=_sc_mesh(),
        out_type=tuple(jax.ShapeDtypeStruct((N, 128), F32) for _ in range(4)),
        scratch_types=[
            pltpu.VMEM((SCH,), jnp.int32),
            pltpu.VMEM((SCH, 128), F32),
            pltpu.VMEM((80, 128), F32),
            pltpu.VMEM_SHARED((N, 128), F32),
        ],
    )
    def _scatter(m0, m1, m2, m3, dst_hbm, o0, o1, o2, o3, idx_v, m_v, z_v, acc):
        c = lax.axis_index("c")
        s = lax.axis_index("s")

        @pl.loop(0, 80)
        def _(r):
            @pl.loop(0, 8)
            def _(cc):
                z_v[pl.ds(r, 1), pl.ds(cc * 16, 16)] = jnp.zeros((1, 16), F32)

        r0 = s * ZROWS
        nblk = jnp.where(s == NS - 1, 5, 8)

        def strip_pass(m_hbm, o_hbm):
            @pl.loop(0, nblk)
            def _(k):
                pltpu.sync_copy(z_v, acc.at[pl.ds(r0 + k * 80, 80)])

            plsc.subcore_barrier()

            ebase = s * EPS_SC

            @pl.loop(0, SNCH)
            def _(k):
                off = ebase + k * SCH
                pltpu.sync_copy(dst_hbm.at[pl.ds(off, SCH)], idx_v)
                pltpu.sync_copy(m_hbm.at[pl.ds(off, SCH)], m_v)
                pltpu.sync_copy(m_v, acc.at[idx_v], add=True)

            plsc.subcore_barrier()

            @pl.loop(0, nblk)
            def _(k):
                pltpu.sync_copy(
                    acc.at[pl.ds(r0 + k * 80, 80)], o_hbm.at[pl.ds(r0 + k * 80, 80)]
                )

            plsc.subcore_barrier()

        @pl.when(c == 0)
        def _():
            strip_pass(m0, o0)
            strip_pass(m1, o1)

        @pl.when(c == 1)
        def _():
            strip_pass(m2, o2)
            strip_pass(m3, o3)

    return _scatter


def _scatter_add(m0, m1, m2, m3, dst):
    return _scatter_add_kernel()(m0, m1, m2, m3, dst)


def _mp_layer(h, src, dst, edge_attr, Wm, bm, Wu, bu):
    wcat = jnp.concatenate([Wm[:H], Wm[H : 2 * H], Wu[:H]], axis=1)
    a, b, u = _proj3(h, wcat)
    g1, g2 = _gather_ab(a, b, src, dst)
    strips = _msg(g1, g2, edge_attr, Wm[2 * H :], bm)
    aggr_strips = _scatter_add(strips[0], strips[1], strips[2], strips[3], dst)
    return u, aggr_strips, Wu[H:], bu


def kernel(x, edge_index, batch_idx, edge_attr, W_emb, b_emb, Wm0, bm0, Wu0, bu0, g0, be0, Wm1, bm1, Wu1, bu1, g1, be1, Wo1, bo1, Wo2, bo2):
    src = edge_index[0]
    dst = edge_index[1]

    h = _embed(x, W_emb, b_emb)

    u, aggr_strips, wub, bu = _mp_layer(h, src, dst, edge_attr, Wm0, bm0, Wu0, bu0)
    h = _update(u, aggr_strips, wub, bu, g0, be0, h)

    u, aggr_strips, wub, bu = _mp_layer(h, src, dst, edge_attr, Wm1, bm1, Wu1, bu1)
    h = _update(u, aggr_strips, wub, bu, g1, be1, h)

    pooled = _pool(h, batch_idx)
    return _head(pooled, Wo1, bo1, Wo2, bo2)


# speedup vs baseline: 3.4689x; 1.9803x over previous
import functools

import jax
import jax.numpy as jnp
from jax import lax
from jax.experimental import pallas as pl
from jax.experimental.pallas import tpu as pltpu
from jax.experimental.pallas import tpu_sc as plsc

N = 10000
E = 160000
DF = 256
H = 512
DE = 16
OUT = 128
G = 64
EPS = 1e-5

F32 = jnp.float32
I32 = jnp.int32
HP = H // 2

NB = 1000
NGRID = N // NB
EB = 1000
EGRID = E // EB

NC = 2
NS = 16
NW = NC * NS


def _embed_body(x_ref, w_ref, b_ref, o_ref):
    o_ref[...] = (
        jnp.dot(x_ref[...], w_ref[...], preferred_element_type=F32) + b_ref[...]
    )


def _embed(x, w, b):
    return pl.pallas_call(
        _embed_body,
        grid=(NGRID,),
        in_specs=[
            pl.BlockSpec((NB, DF), lambda i: (i, 0)),
            pl.BlockSpec((DF, H), lambda i: (0, 0)),
            pl.BlockSpec((1, H), lambda i: (0, 0)),
        ],
        out_specs=pl.BlockSpec((NB, H), lambda i: (i, 0)),
        out_shape=jax.ShapeDtypeStruct((N, H), F32),
    )(x, w, b.reshape(1, H))


def _pack_bf16(lo_f32, hi_f32):
    def rne_hi16(x):
        u = lax.bitcast_convert_type(x, jnp.uint32)
        return (u + 0x7FFF + ((u >> 16) & 1)) & jnp.uint32(0xFFFF0000)

    lo = rne_hi16(lo_f32) >> 16
    hi = rne_hi16(hi_f32)
    return lax.bitcast_convert_type(hi | lo, I32)


def _proj3_body(h_ref, w_ref, a_ref, b_ref, u_ref):
    r = jnp.dot(h_ref[...], w_ref[...], preferred_element_type=F32)
    a_ref[...] = _pack_bf16(r[:, :HP], r[:, HP:H])
    b_ref[...] = _pack_bf16(r[:, H : H + HP], r[:, H + HP : 2 * H])
    u_ref[...] = r[:, 2 * H :]


def _proj3(h, wcat):
    return pl.pallas_call(
        _proj3_body,
        grid=(NGRID,),
        in_specs=[
            pl.BlockSpec((NB, H), lambda i: (i, 0)),
            pl.BlockSpec((H, 3 * H), lambda i: (0, 0)),
        ],
        out_specs=[
            pl.BlockSpec((NB, HP), lambda i: (i, 0)),
            pl.BlockSpec((NB, HP), lambda i: (i, 0)),
            pl.BlockSpec((NB, H), lambda i: (i, 0)),
        ],
        out_shape=[
            jax.ShapeDtypeStruct((N, HP), I32),
            jax.ShapeDtypeStruct((N, HP), I32),
            jax.ShapeDtypeStruct((N, H), F32),
        ],
    )(h, wcat)


def _unpack_lo(p):
    return lax.bitcast_convert_type(p << 16, F32)


def _unpack_hi(p):
    u = lax.bitcast_convert_type(p, jnp.uint32) & jnp.uint32(0xFFFF0000)
    return lax.bitcast_convert_type(u, F32)


def _msg_body(g1_ref, g2_ref, ea_ref, wb_ref, bm_ref, m0, m1, m2, m3):
    g1 = g1_ref[...]
    g2 = g2_ref[...]
    ep = jnp.dot(ea_ref[...], wb_ref[...], preferred_element_type=F32) + bm_ref[...]
    lo = jnp.maximum(_unpack_lo(g1) + _unpack_lo(g2) + ep[:, :HP], 0.0)
    hi = jnp.maximum(_unpack_hi(g1) + _unpack_hi(g2) + ep[:, HP:], 0.0)
    m0[...] = lo[:, :128]
    m1[...] = lo[:, 128:]
    m2[...] = hi[:, :128]
    m3[...] = hi[:, 128:]


def _msg(g1, g2, ea, wb, bm):
    strip = jax.ShapeDtypeStruct((E, 128), F32)
    return pl.pallas_call(
        _msg_body,
        grid=(EGRID,),
        in_specs=[
            pl.BlockSpec((EB, HP), lambda i: (i, 0)),
            pl.BlockSpec((EB, HP), lambda i: (i, 0)),
            pl.BlockSpec((EB, DE), lambda i: (i, 0)),
            pl.BlockSpec((DE, H), lambda i: (0, 0)),
            pl.BlockSpec((1, H), lambda i: (0, 0)),
        ],
        out_specs=[pl.BlockSpec((EB, 128), lambda i: (i, 0)) for _ in range(4)],
        out_shape=[strip, strip, strip, strip],
    )(g1, g2, ea, wb, bm.reshape(1, H))


def _update_body(u_ref, a0, a1, a2, a3, wub_ref, bu_ref, g_ref, be_ref, hp_ref, o_ref):
    aggr = jnp.concatenate([a0[...], a1[...], a2[...], a3[...]], axis=-1)
    t = (
        u_ref[...]
        + jnp.dot(aggr, wub_ref[...], preferred_element_type=F32)
        + bu_ref[...]
    )
    mu = jnp.mean(t, axis=-1, keepdims=True)
    var = jnp.mean((t - mu) * (t - mu), axis=-1, keepdims=True)
    o_ref[...] = (t - mu) * lax.rsqrt(var + EPS) * g_ref[...] + be_ref[...] + hp_ref[...]


def _update(u, aggr_strips, wub, bu, g, be, h_prev):
    a0, a1, a2, a3 = aggr_strips
    return pl.pallas_call(
        _update_body,
        grid=(NGRID,),
        in_specs=[
            pl.BlockSpec((NB, H), lambda i: (i, 0)),
            pl.BlockSpec((NB, 128), lambda i: (i, 0)),
            pl.BlockSpec((NB, 128), lambda i: (i, 0)),
            pl.BlockSpec((NB, 128), lambda i: (i, 0)),
            pl.BlockSpec((NB, 128), lambda i: (i, 0)),
            pl.BlockSpec((H, H), lambda i: (0, 0)),
            pl.BlockSpec((1, H), lambda i: (0, 0)),
            pl.BlockSpec((1, H), lambda i: (0, 0)),
            pl.BlockSpec((1, H), lambda i: (0, 0)),
            pl.BlockSpec((NB, H), lambda i: (i, 0)),
        ],
        out_specs=pl.BlockSpec((NB, H), lambda i: (i, 0)),
        out_shape=jax.ShapeDtypeStruct((N, H), F32),
    )(u, a0, a1, a2, a3, wub, bu.reshape(1, H), g.reshape(1, H), be.reshape(1, H), h_prev)


def _pool_body(h_ref, bidx_ref, o_ref):
    i = pl.program_id(0)
    b = bidx_ref[0, 0, :]
    oh = (lax.broadcasted_iota(jnp.int32, (G, NB), 0) == b[None, :]).astype(F32)
    part = jnp.dot(oh, h_ref[...], preferred_element_type=F32)

    @pl.when(i == 0)
    def _():
        o_ref[...] = part

    @pl.when(i > 0)
    def _():
        o_ref[...] += part


def _pool(h, batch_idx):
    bidx3 = batch_idx.reshape(NGRID, 1, NB)
    return pl.pallas_call(
        _pool_body,
        grid=(NGRID,),
        in_specs=[
            pl.BlockSpec((NB, H), lambda i: (i, 0)),
            pl.BlockSpec((1, 1, NB), lambda i: (i, 0, 0)),
        ],
        out_specs=pl.BlockSpec((G, H), lambda i: (0, 0)),
        out_shape=jax.ShapeDtypeStruct((G, H), F32),
    )(h, bidx3)


def _head_body(p_ref, w1_ref, b1_ref, w2_ref, b2_ref, o_ref):
    t = jnp.dot(p_ref[...], w1_ref[...], preferred_element_type=F32) + b1_ref[...]
    o_ref[...] = jnp.dot(t, w2_ref[...], preferred_element_type=F32) + b2_ref[...]


def _head(p, w1, b1, w2, b2):
    return pl.pallas_call(
        _head_body,
        in_specs=[
            pl.BlockSpec((G, H), lambda: (0, 0)),
            pl.BlockSpec((H, H), lambda: (0, 0)),
            pl.BlockSpec((1, H), lambda: (0, 0)),
            pl.BlockSpec((H, OUT), lambda: (0, 0)),
            pl.BlockSpec((1, OUT), lambda: (0, 0)),
        ],
        out_specs=pl.BlockSpec((G, OUT), lambda: (0, 0)),
        out_shape=jax.ShapeDtypeStruct((G, OUT), F32),
    )(p, w1, b1.reshape(1, H), w2, b2.reshape(1, OUT))


GW = 128


@functools.lru_cache(maxsize=None)
def _sc_mesh():
    return plsc.VectorSubcoreMesh(core_axis_name="c", subcore_axis_name="s")


@functools.lru_cache(maxsize=None)
def _gather_ab_kernel():
    @functools.partial(
        pl.kernel,
        mesh=_sc_mesh(),
        out_type=(
            jax.ShapeDtypeStruct((E, HP), I32),
            jax.ShapeDtypeStruct((E, HP), I32),
        ),
    )
    def _gather_ab(a_hbm, b_hbm, src3_hbm, dst3_hbm, g1_hbm, g2_hbm):
        def one_table(tab, idx3, out):
            def body(i_vmem, o_vmem):
                pltpu.sync_copy(tab.at[i_vmem.at[0, 0]], o_vmem)

            pltpu.emit_pipeline(
                body,
                grid=(E // GW,),
                in_specs=[pl.BlockSpec((1, 1, GW), lambda i: (i, 0, 0))],
                out_specs=[pl.BlockSpec((GW, HP), lambda i: (i, 0))],
                core_axis_name=("c", "s"),
                dimension_semantics=(pltpu.PARALLEL,),
            )(idx3, out)

        one_table(a_hbm, src3_hbm, g1_hbm)
        one_table(b_hbm, dst3_hbm, g2_hbm)

    return _gather_ab


def _gather_ab(a, b, src, dst):
    return _gather_ab_kernel()(
        a, b, src.reshape(E // GW, 1, GW), dst.reshape(E // GW, 1, GW)
    )


SCH = 80
ZROWS = 640


@functools.lru_cache(maxsize=None)
def _scatter_add_kernel():
    @functools.partial(
        pl.kernel,
        mesh=_sc_mesh(),
        out_type=tuple(jax.ShapeDtypeStruct((N, 128), F32) for _ in range(4)),
        scratch_types=[
            pltpu.VMEM((80, 128), F32),
            pltpu.VMEM_SHARED((N, 128), F32),
        ],
    )
    def _scatter(m0, m1, m2, m3, dst3_hbm, o0, o1, o2, o3, z_v, acc):
        c = lax.axis_index("c")
        s = lax.axis_index("s")

        @pl.loop(0, 80)
        def _(r):
            @pl.loop(0, 8)
            def _(cc):
                z_v[pl.ds(r, 1), pl.ds(cc * 16, 16)] = jnp.zeros((1, 16), F32)

        r0 = s * ZROWS

        def body(i_vmem, m_vmem):
            pltpu.sync_copy(m_vmem, acc.at[i_vmem.at[0, 0]], add=True)

        def strip_pass(m_hbm, o_hbm):
            @pl.when(s < NS - 1)
            def _():
                @pl.loop(0, 8)
                def _(k):
                    pltpu.sync_copy(z_v, acc.at[pl.ds(r0 + k * 80, 80)])

            @pl.when(s == NS - 1)
            def _():
                @pl.loop(0, 5)
                def _(k):
                    pltpu.sync_copy(z_v, acc.at[pl.ds(r0 + k * 80, 80)])

            plsc.subcore_barrier()

            pltpu.emit_pipeline(
                body,
                grid=(E // SCH,),
                in_specs=[
                    pl.BlockSpec((1, 1, SCH), lambda i: (i, 0, 0)),
                    pl.BlockSpec((SCH, 128), lambda i: (i, 0)),
                ],
                out_specs=[],
                core_axis_name="s",
                dimension_semantics=(pltpu.PARALLEL,),
            )(dst3_hbm, m_hbm)

            plsc.subcore_barrier()

            @pl.when(s < NS - 1)
            def _():
                pltpu.sync_copy(acc.at[pl.ds(r0, ZROWS)], o_hbm.at[pl.ds(r0, ZROWS)])

            @pl.when(s == NS - 1)
            def _():
                pltpu.sync_copy(acc.at[pl.ds(r0, 400)], o_hbm.at[pl.ds(r0, 400)])

            plsc.subcore_barrier()

        @pl.when(c == 0)
        def _():
            strip_pass(m0, o0)
            strip_pass(m1, o1)

        @pl.when(c == 1)
        def _():
            strip_pass(m2, o2)
            strip_pass(m3, o3)

    return _scatter


def _scatter_add(m0, m1, m2, m3, dst):
    return _scatter_add_kernel()(m0, m1, m2, m3, dst.reshape(E // SCH, 1, SCH))


def _mp_layer(h, src, dst, edge_attr, Wm, bm, Wu, bu):
    wcat = jnp.concatenate([Wm[:H], Wm[H : 2 * H], Wu[:H]], axis=1)
    a, b, u = _proj3(h, wcat)
    g1, g2 = _gather_ab(a, b, src, dst)
    strips = _msg(g1, g2, edge_attr, Wm[2 * H :], bm)
    aggr_strips = _scatter_add(strips[0], strips[1], strips[2], strips[3], dst)
    return u, aggr_strips, Wu[H:], bu


def kernel(x, edge_index, batch_idx, edge_attr, W_emb, b_emb, Wm0, bm0, Wu0, bu0, g0, be0, Wm1, bm1, Wu1, bu1, g1, be1, Wo1, bo1, Wo2, bo2):
    src = edge_index[0]
    dst = edge_index[1]

    h = _embed(x, W_emb, b_emb)

    u, aggr_strips, wub, bu = _mp_layer(h, src, dst, edge_attr, Wm0, bm0, Wu0, bu0)
    h = _update(u, aggr_strips, wub, bu, g0, be0, h)

    u, aggr_strips, wub, bu = _mp_layer(h, src, dst, edge_attr, Wm1, bm1, Wu1, bu1)
    h = _update(u, aggr_strips, wub, bu, g1, be1, h)

    pooled = _pool(h, batch_idx)
    return _head(pooled, Wo1, bo1, Wo2, bo2)
